# baseline (speedup 1.0000x reference)
import jax
import jax.numpy as jnp
from jax.experimental import pallas as pl
from jax.experimental.pallas import tpu as pltpu

_B = 128
_CP = 256
_CC = 512
_HP = 28
_HC = 14
_DEPTH = 6
_ODEPTH = 2


def _stream_kernel(c_hbm, p_hbm, o_hbm, cbuf, pbuf, obuf, acc, wv,
                   csem, psem, osem):
    for s in range(_DEPTH):
        pltpu.make_async_copy(
            c_hbm.at[pl.ds(s, 1)], cbuf.at[s], csem.at[s]
        ).start()
    acc[...] = jnp.zeros_like(acc)

    def cbody(k, carry):
        slot = jax.lax.rem(k, _DEPTH)
        pltpu.make_async_copy(
            cbuf.at[slot], cbuf.at[slot], csem.at[slot]
        ).wait()
        m = jnp.where(cbuf[slot] > 0.0, 1.0, 0.0)
        acc[...] += jnp.sum(m, axis=(0, 1))

        nxt = k + _DEPTH

        @pl.when(nxt < _HC)
        def _():
            pltpu.make_async_copy(
                c_hbm.at[pl.ds(nxt, 1)], cbuf.at[slot], csem.at[slot]
            ).start()

        @pl.when(nxt >= _HC)
        def _():
            j = nxt - _HC
            pltpu.make_async_copy(
                p_hbm.at[pl.ds(j, 1)],
                pbuf.at[jax.lax.rem(j, _DEPTH)],
                psem.at[jax.lax.rem(j, _DEPTH)],
            ).start()

        return carry

    jax.lax.fori_loop(0, _HC, cbody, 0)

    tot = jnp.sum(acc[...], axis=1, keepdims=True)
    wv[...] = jnp.broadcast_to(tot, wv.shape)

    def pbody(j, carry):
        slot = jax.lax.rem(j, _DEPTH)
        oslot = jax.lax.rem(j, _ODEPTH)
        pltpu.make_async_copy(
            pbuf.at[slot], pbuf.at[slot], psem.at[slot]
        ).wait()

        @pl.when(j >= _ODEPTH)
        def _():
            pltpu.make_async_copy(
                obuf.at[oslot], obuf.at[oslot], osem.at[oslot]
            ).wait()

        x = pbuf[slot]
        sel = jnp.where(x > 0.0, wv[...][None, None], 0.0)
        obuf[oslot] = jnp.sum(sel, axis=2)
        pltpu.make_async_copy(
            obuf.at[oslot], o_hbm.at[pl.ds(j, 1)], osem.at[oslot]
        ).start()

        nxt = j + _DEPTH

        @pl.when(nxt < _HP)
        def _():
            pltpu.make_async_copy(
                p_hbm.at[pl.ds(nxt, 1)], pbuf.at[slot], psem.at[slot]
            ).start()

        return carry

    jax.lax.fori_loop(0, _HP, pbody, 0)

    for s in range(_ODEPTH):
        pltpu.make_async_copy(
            obuf.at[s], obuf.at[s], osem.at[s]
        ).wait()


def kernel(prev_x, curr_x):
    pv = jnp.transpose(prev_x, (2, 3, 0, 1))
    cv = jnp.transpose(curr_x, (2, 3, 0, 1))

    out = pl.pallas_call(
        _stream_kernel,
        in_specs=[
            pl.BlockSpec(memory_space=pl.ANY),
            pl.BlockSpec(memory_space=pl.ANY),
        ],
        out_specs=pl.BlockSpec(memory_space=pl.ANY),
        out_shape=jax.ShapeDtypeStruct((_HP, _HP, _CP), jnp.float32),
        scratch_shapes=[
            pltpu.VMEM((_DEPTH, 1, _HC, _B, _CC), jnp.float32),
            pltpu.VMEM((_DEPTH, 1, _HP, _B, _CP), jnp.float32),
            pltpu.VMEM((_ODEPTH, 1, _HP, _CP), jnp.float32),
            pltpu.VMEM((_B, _CC), jnp.float32),
            pltpu.VMEM((_B, _CP), jnp.float32),
            pltpu.SemaphoreType.DMA((_DEPTH,)),
            pltpu.SemaphoreType.DMA((_DEPTH,)),
            pltpu.SemaphoreType.DMA((_ODEPTH,)),
        ],
        compiler_params=pltpu.CompilerParams(
            vmem_limit_bytes=50 * 1024 * 1024,
        ),
    )(cv, pv)

    return jnp.transpose(out, (2, 0, 1))


# speedup vs baseline: 1.0143x; 1.0143x over previous
import jax
import jax.numpy as jnp
from jax.experimental import pallas as pl
from jax.experimental.pallas import tpu as pltpu

_B = 128
_CP = 256
_CC = 512
_HP = 28
_HC = 14
_SPLIT = 2
_NCC = _HC * _SPLIT
_NPC = _HP * _SPLIT
_CW = _HC // _SPLIT
_PW = _HP // _SPLIT
_DEPTH = 6
_ODEPTH = 2


def _stream_kernel(c_hbm, p_hbm, o_hbm, cbuf, pbuf, obuf, acc, wv,
                   csem, psem, osem):
    for s in range(_DEPTH):
        pltpu.make_async_copy(
            c_hbm.at[pl.ds(s, 1)], cbuf.at[s], csem.at[s]
        ).start()
    acc[...] = jnp.zeros_like(acc)

    def cbody(k, carry):
        slot = jax.lax.rem(k, _DEPTH)
        pltpu.make_async_copy(
            cbuf.at[slot], cbuf.at[slot], csem.at[slot]
        ).wait()
        m = jnp.where(cbuf[slot] > 0.0, 1.0, 0.0)
        acc[...] += jnp.sum(m, axis=(0, 1))

        nxt = k + _DEPTH

        @pl.when(nxt < _NCC)
        def _():
            pltpu.make_async_copy(
                c_hbm.at[pl.ds(nxt, 1)], cbuf.at[slot], csem.at[slot]
            ).start()

        @pl.when(nxt >= _NCC)
        def _():
            j = nxt - _NCC
            pltpu.make_async_copy(
                p_hbm.at[pl.ds(j, 1)],
                pbuf.at[jax.lax.rem(j, _DEPTH)],
                psem.at[jax.lax.rem(j, _DEPTH)],
            ).start()

        return carry

    jax.lax.fori_loop(0, _NCC, cbody, 0)

    tot = jnp.sum(acc[...], axis=1, keepdims=True)
    wv[...] = jnp.broadcast_to(tot, wv.shape)

    def pbody(j, carry):
        slot = jax.lax.rem(j, _DEPTH)
        oslot = jax.lax.rem(j, _ODEPTH)
        pltpu.make_async_copy(
            pbuf.at[slot], pbuf.at[slot], psem.at[slot]
        ).wait()

        @pl.when(j >= _ODEPTH)
        def _():
            pltpu.make_async_copy(
                obuf.at[oslot], obuf.at[oslot], osem.at[oslot]
            ).wait()

        x = pbuf[slot]
        sel = jnp.where(x > 0.0, wv[...][None, None], 0.0)
        obuf[oslot] = jnp.sum(sel, axis=2)
        pltpu.make_async_copy(
            obuf.at[oslot], o_hbm.at[pl.ds(j, 1)], osem.at[oslot]
        ).start()

        nxt = j + _DEPTH

        @pl.when(nxt < _NPC)
        def _():
            pltpu.make_async_copy(
                p_hbm.at[pl.ds(nxt, 1)], pbuf.at[slot], psem.at[slot]
            ).start()

        return carry

    jax.lax.fori_loop(0, _NPC, pbody, 0)

    for s in range(_ODEPTH):
        pltpu.make_async_copy(
            obuf.at[s], obuf.at[s], osem.at[s]
        ).wait()


def kernel(prev_x, curr_x):
    pv = jnp.transpose(prev_x, (2, 3, 0, 1)).reshape(_NPC, _PW, _B, _CP)
    cv = jnp.transpose(curr_x, (2, 3, 0, 1)).reshape(_NCC, _CW, _B, _CC)

    out = pl.pallas_call(
        _stream_kernel,
        in_specs=[
            pl.BlockSpec(memory_space=pl.ANY),
            pl.BlockSpec(memory_space=pl.ANY),
        ],
        out_specs=pl.BlockSpec(memory_space=pl.ANY),
        out_shape=jax.ShapeDtypeStruct((_NPC, _PW, _CP), jnp.float32),
        scratch_shapes=[
            pltpu.VMEM((_DEPTH, 1, _CW, _B, _CC), jnp.float32),
            pltpu.VMEM((_DEPTH, 1, _PW, _B, _CP), jnp.float32),
            pltpu.VMEM((_ODEPTH, 1, _PW, _CP), jnp.float32),
            pltpu.VMEM((_B, _CC), jnp.float32),
            pltpu.VMEM((_B, _CP), jnp.float32),
            pltpu.SemaphoreType.DMA((_DEPTH,)),
            pltpu.SemaphoreType.DMA((_DEPTH,)),
            pltpu.SemaphoreType.DMA((_ODEPTH,)),
        ],
        compiler_params=pltpu.CompilerParams(
            vmem_limit_bytes=50 * 1024 * 1024,
        ),
    )(cv, pv)

    return jnp.transpose(out.reshape(_HP, _HP, _CP), (2, 0, 1))
